# Initial kernel scaffold; baseline (speedup 1.0000x reference)
import jax
import jax.numpy as jnp
from jax.experimental import pallas as pl


def kernel(input, adj, M, W, c1, c2):
    raise NotImplementedError("write your pallas kernel here")


# speedup vs baseline: 3.0617x; 3.0617x over previous
import jax
import jax.numpy as jnp
from jax.experimental import pallas as pl
from jax.experimental.pallas import tpu as pltpu

_ALPHA = 0.2
_NEG = -1e9


def _proj_kernel(x_ref, w_ref, c1_ref, c2_ref, h_ref, s1_ref, s2e_ref):
    h = jnp.dot(x_ref[...], w_ref[...], preferred_element_type=jnp.float32)
    h_ref[...] = h
    s1_ref[...] = jnp.dot(h, c1_ref[...], preferred_element_type=jnp.float32)
    s2 = jnp.dot(h, c2_ref[...], preferred_element_type=jnp.float32)
    nz = jnp.sum(h, axis=1, keepdims=True) != 0.0
    s2e_ref[...] = jnp.where(nz, s2, _NEG)


def _att_kernel(adj_ref, s1_ref, s2e_ref, rv_ref, h_ref, out_ref):
    s1 = s1_ref[...]
    t = s1 + s2e_ref[...]
    e = jnp.maximum(t, _ALPHA * t)
    w = jnp.where(adj_ref[...] > 0.0, jnp.exp(e - s1), 0.0)
    w = w * rv_ref[...]
    num = jnp.dot(w, h_ref[...], preferred_element_type=jnp.float32)
    den = jnp.sum(w, axis=1, keepdims=True)
    hp = jnp.where(den > 0.0, num / den, 0.0)
    out_ref[...] = jnp.where(hp > 0.0, hp, jnp.exp(hp) - 1.0)


def kernel(input, adj, M, W, c1, c2):
    N, Fin = input.shape
    Fout = W.shape[1]

    h, s1, s2e = pl.pallas_call(
        _proj_kernel,
        out_shape=[
            jax.ShapeDtypeStruct((N, Fout), jnp.float32),
            jax.ShapeDtypeStruct((N, 1), jnp.float32),
            jax.ShapeDtypeStruct((N, 1), jnp.float32),
        ],
    )(input, W, c1, c2)

    s2row = s2e.reshape(1, N)
    rv = (jnp.arange(N) < M).astype(jnp.float32).reshape(N, 1)

    for cand in (200, 100, 40, 8, 1):
        if N % cand == 0:
            TM = cand
            break

    out = pl.pallas_call(
        _att_kernel,
        grid=(N // TM,),
        in_specs=[
            pl.BlockSpec((TM, N), lambda i: (i, 0)),
            pl.BlockSpec((TM, 1), lambda i: (i, 0)),
            pl.BlockSpec((1, N), lambda i: (0, 0)),
            pl.BlockSpec((TM, 1), lambda i: (i, 0)),
            pl.BlockSpec((N, Fout), lambda i: (0, 0)),
        ],
        out_specs=pl.BlockSpec((TM, Fout), lambda i: (i, 0)),
        out_shape=jax.ShapeDtypeStruct((N, Fout), jnp.float32),
        compiler_params=pltpu.CompilerParams(
            dimension_semantics=("parallel",),
        ),
    )(adj, s1, s2row, rv, h)
    return out
